# baseline (speedup 1.0000x reference)
import functools

import jax
import jax.numpy as jnp
from jax import lax
from jax.experimental import pallas as pl
from jax.experimental.pallas import tpu as pltpu
from jax.experimental.pallas import tpu_sc as plsc

_N = 10000
_E = 320000
_D_IN = 128
_D_HID = 8
_D_OUT = 16

_SC_NC = 1
_SC_NS = 16
_NW = _SC_NC * _SC_NS
_ECHUNK = 256
_KPT = 80
_E_PAD = _KPT * _NW * _ECHUNK
_RPT = 632
_ACC_N = _RPT * _SC_NS
_W = 16


_NB = 8
_G = _KPT // _NB
_ZR = 80
_RCH = [(k * _ZR, min(_ZR, _RPT - k * _ZR)) for k in range(_NB)]


def _sc_segsum_body(src_hbm, dst_hbm, tab_hbm, out_hbm,
                    src_v, dst_v, rows_v, slice_v, acc_sh, gsem, ssem, csem):
    c = lax.axis_index("c")
    s = lax.axis_index("s")
    wid = s * _SC_NC + c

    pltpu.async_copy(src_hbm.at[wid], src_v, csem.at[0])
    pltpu.async_copy(dst_hbm.at[wid], dst_v, csem.at[1])

    def _zero(i, carry):
        slice_v[i] = jnp.zeros((16,), jnp.float32)
        return carry
    lax.fori_loop(0, _ZR, _zero, 0)

    pltpu.make_async_copy(src_hbm.at[wid], src_v, csem.at[0]).wait()
    for b in range(_NB):
        pltpu.async_copy(tab_hbm.at[src_v.at[b]],
                         rows_v.at[b], gsem.at[b])

    for k, (r0, nr) in enumerate(_RCH):
        pltpu.async_copy(slice_v.at[pl.ds(0, nr)],
                         acc_sh.at[pl.ds(s * _RPT + r0, nr)], ssem.at[k])
    for k, (r0, nr) in enumerate(_RCH):
        pltpu.make_async_copy(slice_v.at[pl.ds(0, nr)],
                              acc_sh.at[pl.ds(s * _RPT + r0, nr)],
                              ssem.at[k]).wait()
    plsc.subcore_barrier()
    pltpu.make_async_copy(dst_hbm.at[wid], dst_v, csem.at[1]).wait()

    def _super(gg, carry):
        j0 = gg * _NB
        for b in range(_NB):
            pltpu.make_async_copy(
                tab_hbm.at[src_v.at[j0 + b]],
                rows_v.at[b], gsem.at[b]).wait()
            pltpu.async_copy(
                rows_v.at[b], acc_sh.at[dst_v.at[j0 + b]],
                ssem.at[b], add=True)
        for b in range(_NB):
            pltpu.make_async_copy(
                rows_v.at[b], acc_sh.at[dst_v.at[j0 + b]],
                ssem.at[b]).wait()

            @pl.when(gg + 1 < _G)
            def _():
                pltpu.async_copy(
                    tab_hbm.at[src_v.at[j0 + _NB + b]],
                    rows_v.at[b], gsem.at[b])
        return carry
    lax.fori_loop(0, _G, _super, 0)
    plsc.subcore_barrier()

    for k, (r0, nr) in enumerate(_RCH):
        pltpu.async_copy(acc_sh.at[pl.ds(s * _RPT + r0, nr)],
                         slice_v.at[pl.ds(r0, nr)], gsem.at[k])
    for k, (r0, nr) in enumerate(_RCH):
        pltpu.make_async_copy(acc_sh.at[pl.ds(s * _RPT + r0, nr)],
                              slice_v.at[pl.ds(r0, nr)], gsem.at[k]).wait()
        pltpu.async_copy(slice_v.at[pl.ds(r0, nr)],
                         out_hbm.at[c].at[pl.ds(s * _RPT + r0, nr)],
                         ssem.at[k])
    for k, (r0, nr) in enumerate(_RCH):
        pltpu.make_async_copy(slice_v.at[pl.ds(r0, nr)],
                              out_hbm.at[c].at[pl.ds(s * _RPT + r0, nr)],
                              ssem.at[k]).wait()


@functools.cache
def _sc_segsum():
    return pl.kernel(
        _sc_segsum_body,
        out_type=jax.ShapeDtypeStruct((_SC_NC, _ACC_N, _W), jnp.float32),
        mesh=plsc.VectorSubcoreMesh(core_axis_name="c", subcore_axis_name="s",
                                    num_cores=_SC_NC, num_subcores=_SC_NS),
        scratch_types=[
            pltpu.VMEM((_KPT, _ECHUNK), jnp.int32),
            pltpu.VMEM((_KPT, _ECHUNK), jnp.int32),
            pltpu.VMEM((_NB, _ECHUNK, _W), jnp.float32),
            pltpu.VMEM((_RPT, _W), jnp.float32),
            pltpu.VMEM_SHARED((_ACC_N, _W), jnp.float32),
            pltpu.SemaphoreType.DMA((_NB,)),
            pltpu.SemaphoreType.DMA((_NB,)),
            pltpu.SemaphoreType.DMA((2,)),
        ],
        compiler_params=pltpu.CompilerParams(use_tc_tiling_on_sc=False),
    )


def _tc1_body(x_ref, wl_ref, wr_ref, t1_ref, y1r_ref):
    xb = x_ref[...]
    yl = lax.dot_general(xb, wl_ref[...], (((1,), (1,)), ((), ())),
                         preferred_element_type=jnp.float32)
    yr = lax.dot_general(xb, wr_ref[...], (((1,), (1,)), ((), ())),
                         preferred_element_type=jnp.float32)
    ones = jnp.ones((xb.shape[0], 1), jnp.float32)
    zeros = jnp.zeros((xb.shape[0], _W - _D_HID - 1), jnp.float32)
    t1_ref[...] = jnp.concatenate([yl, ones, zeros], axis=1)
    y1r_ref[...] = yr


def _tc2_body(p_ref, y1r_ref, b1l_ref, w2l_ref, w2r_ref, t2_ref, y2r_ref):
    stot = p_ref[0, :_N]
    sums = stot[:, 0:_D_HID]
    cnt = stot[:, _D_HID:_D_HID + 1]
    inv = 1.0 / jnp.maximum(cnt, 1.0)
    h = jnp.maximum(sums * inv + b1l_ref[...] + y1r_ref[...], 0.0)
    t2_ref[...] = lax.dot_general(h, w2l_ref[...], (((1,), (1,)), ((), ())),
                                  preferred_element_type=jnp.float32)
    y2r_ref[...] = lax.dot_general(h, w2r_ref[...], (((1,), (1,)), ((), ())),
                                   preferred_element_type=jnp.float32)


def _tc3_body(q_ref, p_ref, y2r_ref, b2l_ref, out_ref):
    qsum = q_ref[0, :_N]
    cnt = p_ref[0, :_N, _D_HID:_D_HID + 1]
    inv = 1.0 / jnp.maximum(cnt, 1.0)
    out_ref[...] = qsum * inv + b2l_ref[...] + y2r_ref[...]


def kernel(x, edge_index, W1l, b1l, W1r, W2l, b2l, W2r):
    pad = _E_PAD - _E
    src = jnp.concatenate([edge_index[0], jnp.zeros((pad,), jnp.int32)])
    dst = jnp.concatenate([edge_index[1], jnp.full((pad,), _N, jnp.int32)])
    src = src.reshape(_NW, _KPT, _ECHUNK)
    dst = dst.reshape(_NW, _KPT, _ECHUNK)

    t1, y1r = pl.pallas_call(
        _tc1_body,
        out_shape=[jax.ShapeDtypeStruct((_N, _W), jnp.float32),
                   jax.ShapeDtypeStruct((_N, _D_HID), jnp.float32)],
    )(x, W1l, W1r)

    p = _sc_segsum()(src, dst, t1)

    t2, y2r = pl.pallas_call(
        _tc2_body,
        out_shape=[jax.ShapeDtypeStruct((_N, _W), jnp.float32),
                   jax.ShapeDtypeStruct((_N, _D_OUT), jnp.float32)],
    )(p, y1r, b1l.reshape(1, _D_HID), W2l, W2r)

    q = _sc_segsum()(src, dst, t2)

    out = pl.pallas_call(
        _tc3_body,
        out_shape=jax.ShapeDtypeStruct((_N, _D_OUT), jnp.float32),
    )(q, p, y2r, b2l.reshape(1, _D_OUT))
    return out


# speedup vs baseline: 1.6347x; 1.6347x over previous
import functools

import jax
import jax.numpy as jnp
from jax import lax
from jax.experimental import pallas as pl
from jax.experimental.pallas import tpu as pltpu
from jax.experimental.pallas import tpu_sc as plsc

_N = 10000
_E = 320000
_D_IN = 128
_D_HID = 8
_D_OUT = 16

_SC_NC = 2
_SC_NS = 16
_NW = _SC_NC * _SC_NS
_ECHUNK = 256
_KPT = 40
_E_PAD = _KPT * _NW * _ECHUNK
_RPT = 632
_ACC_N = _RPT * _SC_NS
_W = 16


_NB = 8
_G = _KPT // _NB
_ZR = 80
_RCH = [(k * _ZR, min(_ZR, _RPT - k * _ZR)) for k in range(_NB)]


def _sc_segsum_body(src_hbm, dst_hbm, tab_hbm, out_hbm,
                    src_v, dst_v, rows_v, slice_v, tab_sh, acc_sh,
                    gsem, ssem, csem):
    c = lax.axis_index("c")
    s = lax.axis_index("s")
    wid = s * _SC_NC + c

    pltpu.async_copy(src_hbm.at[wid], src_v, csem.at[0])
    pltpu.async_copy(dst_hbm.at[wid], dst_v, csem.at[1])

    pltpu.sync_copy(tab_hbm.at[pl.ds(s * _RPT, _RPT)], slice_v)
    pltpu.sync_copy(slice_v, tab_sh.at[pl.ds(s * _RPT, _RPT)])

    def _zero(i, carry):
        slice_v[i] = jnp.zeros((16,), jnp.float32)
        return carry
    lax.fori_loop(0, _ZR, _zero, 0)

    for k, (r0, nr) in enumerate(_RCH):
        pltpu.async_copy(slice_v.at[pl.ds(0, nr)],
                         acc_sh.at[pl.ds(s * _RPT + r0, nr)], ssem.at[k])
    for k, (r0, nr) in enumerate(_RCH):
        pltpu.make_async_copy(slice_v.at[pl.ds(0, nr)],
                              acc_sh.at[pl.ds(s * _RPT + r0, nr)],
                              ssem.at[k]).wait()
    plsc.subcore_barrier()

    pltpu.make_async_copy(src_hbm.at[wid], src_v, csem.at[0]).wait()
    for b in range(_NB):
        pltpu.async_copy(tab_sh.at[src_v.at[b]],
                         rows_v.at[b], gsem.at[b])
    pltpu.make_async_copy(dst_hbm.at[wid], dst_v, csem.at[1]).wait()

    def _super(gg, carry):
        j0 = gg * _NB
        for b in range(_NB):
            pltpu.make_async_copy(
                tab_sh.at[src_v.at[j0 + b]],
                rows_v.at[b], gsem.at[b]).wait()
            pltpu.async_copy(
                rows_v.at[b], acc_sh.at[dst_v.at[j0 + b]],
                ssem.at[b], add=True)
        for b in range(_NB):
            pltpu.make_async_copy(
                rows_v.at[b], acc_sh.at[dst_v.at[j0 + b]],
                ssem.at[b]).wait()

            @pl.when(gg + 1 < _G)
            def _():
                pltpu.async_copy(
                    tab_sh.at[src_v.at[j0 + _NB + b]],
                    rows_v.at[b], gsem.at[b])
        return carry
    lax.fori_loop(0, _G, _super, 0)
    plsc.subcore_barrier()

    for k, (r0, nr) in enumerate(_RCH):
        pltpu.async_copy(acc_sh.at[pl.ds(s * _RPT + r0, nr)],
                         slice_v.at[pl.ds(r0, nr)], gsem.at[k])
    for k, (r0, nr) in enumerate(_RCH):
        pltpu.make_async_copy(acc_sh.at[pl.ds(s * _RPT + r0, nr)],
                              slice_v.at[pl.ds(r0, nr)], gsem.at[k]).wait()
        pltpu.async_copy(slice_v.at[pl.ds(r0, nr)],
                         out_hbm.at[c].at[pl.ds(s * _RPT + r0, nr)],
                         ssem.at[k])
    for k, (r0, nr) in enumerate(_RCH):
        pltpu.make_async_copy(slice_v.at[pl.ds(r0, nr)],
                              out_hbm.at[c].at[pl.ds(s * _RPT + r0, nr)],
                              ssem.at[k]).wait()


@functools.cache
def _sc_segsum():
    return pl.kernel(
        _sc_segsum_body,
        out_type=jax.ShapeDtypeStruct((_SC_NC, _ACC_N, _W), jnp.float32),
        mesh=plsc.VectorSubcoreMesh(core_axis_name="c", subcore_axis_name="s",
                                    num_cores=_SC_NC, num_subcores=_SC_NS),
        scratch_types=[
            pltpu.VMEM((_KPT, _ECHUNK), jnp.int32),
            pltpu.VMEM((_KPT, _ECHUNK), jnp.int32),
            pltpu.VMEM((_NB, _ECHUNK, _W), jnp.float32),
            pltpu.VMEM((_RPT, _W), jnp.float32),
            pltpu.VMEM_SHARED((_ACC_N, _W), jnp.float32),
            pltpu.VMEM_SHARED((_ACC_N, _W), jnp.float32),
            pltpu.SemaphoreType.DMA((_NB,)),
            pltpu.SemaphoreType.DMA((_NB,)),
            pltpu.SemaphoreType.DMA((2,)),
        ],
        compiler_params=pltpu.CompilerParams(use_tc_tiling_on_sc=False),
    )


def _tc1_body(x_ref, wl_ref, wr_ref, t1_ref, y1r_ref):
    xb = x_ref[...]
    yl = lax.dot_general(xb, wl_ref[...], (((1,), (1,)), ((), ())),
                         preferred_element_type=jnp.float32)
    yr = lax.dot_general(xb, wr_ref[...], (((1,), (1,)), ((), ())),
                         preferred_element_type=jnp.float32)
    ones = jnp.ones((xb.shape[0], 1), jnp.float32)
    zeros = jnp.zeros((xb.shape[0], _W - _D_HID - 1), jnp.float32)
    t1_ref[:_N] = jnp.concatenate([yl, ones, zeros], axis=1)
    t1_ref[_N:] = jnp.zeros((_ACC_N - _N, _W), jnp.float32)
    y1r_ref[...] = yr


def _tc2_body(p_ref, y1r_ref, b1l_ref, w2l_ref, w2r_ref, t2_ref, y2r_ref):
    stot = p_ref[0, :_N] + p_ref[1, :_N]
    sums = stot[:, 0:_D_HID]
    cnt = stot[:, _D_HID:_D_HID + 1]
    inv = 1.0 / jnp.maximum(cnt, 1.0)
    h = jnp.maximum(sums * inv + b1l_ref[...] + y1r_ref[...], 0.0)
    t2_ref[:_N] = lax.dot_general(h, w2l_ref[...], (((1,), (1,)), ((), ())),
                                  preferred_element_type=jnp.float32)
    t2_ref[_N:] = jnp.zeros((_ACC_N - _N, _W), jnp.float32)
    y2r_ref[...] = lax.dot_general(h, w2r_ref[...], (((1,), (1,)), ((), ())),
                                   preferred_element_type=jnp.float32)


def _tc3_body(q_ref, p_ref, y2r_ref, b2l_ref, out_ref):
    qsum = q_ref[0, :_N] + q_ref[1, :_N]
    cnt = (p_ref[0, :_N, _D_HID:_D_HID + 1]
           + p_ref[1, :_N, _D_HID:_D_HID + 1])
    inv = 1.0 / jnp.maximum(cnt, 1.0)
    out_ref[...] = qsum * inv + b2l_ref[...] + y2r_ref[...]


def kernel(x, edge_index, W1l, b1l, W1r, W2l, b2l, W2r):
    pad = _E_PAD - _E
    src = jnp.concatenate([edge_index[0], jnp.zeros((pad,), jnp.int32)])
    dst = jnp.concatenate([edge_index[1], jnp.full((pad,), _N, jnp.int32)])
    src = src.reshape(_NW, _KPT, _ECHUNK)
    dst = dst.reshape(_NW, _KPT, _ECHUNK)

    t1, y1r = pl.pallas_call(
        _tc1_body,
        out_shape=[jax.ShapeDtypeStruct((_ACC_N, _W), jnp.float32),
                   jax.ShapeDtypeStruct((_N, _D_HID), jnp.float32)],
    )(x, W1l, W1r)

    p = _sc_segsum()(src, dst, t1)

    t2, y2r = pl.pallas_call(
        _tc2_body,
        out_shape=[jax.ShapeDtypeStruct((_ACC_N, _W), jnp.float32),
                   jax.ShapeDtypeStruct((_N, _D_OUT), jnp.float32)],
    )(p, y1r, b1l.reshape(1, _D_HID), W2l, W2r)

    q = _sc_segsum()(src, dst, t2)

    out = pl.pallas_call(
        _tc3_body,
        out_shape=jax.ShapeDtypeStruct((_N, _D_OUT), jnp.float32),
    )(q, p, y2r, b2l.reshape(1, _D_OUT))
    return out
